# baseline (speedup 1.0000x reference)
import functools

import jax
import jax.numpy as jnp
from jax import lax
from jax.experimental import pallas as pl
from jax.experimental.pallas import tpu as pltpu
from jax.experimental.pallas import tpu_sc as plsc

T = 2048
H = 768
M = 128
E = 16

B = 128
LOG2B = B.bit_length() - 1
NB = T // B + E
TG = NB * B
NC = 2
NS = 16
NW = NC * NS
TPW = T // NW


def _route_body(seg_ref, pos_ref, be_ref):
    seg = seg_ref[...]
    e_row = lax.broadcasted_iota(jnp.int32, (1, E), 1)
    z = seg == e_row
    zf = z.astype(jnp.bfloat16)
    r = lax.broadcasted_iota(jnp.int32, (T, T), 0)
    c = lax.broadcasted_iota(jnp.int32, (T, T), 1)
    lf = (c < r).astype(jnp.bfloat16)
    rank = lax.dot_general(lf, zf, (((1,), (0,)), ((), ())),
                           preferred_element_type=jnp.float32)
    zf32 = z.astype(jnp.float32)
    hist = jnp.sum(zf32, axis=0, keepdims=True)
    nblk = jnp.ceil(hist * (1.0 / B))
    ir = lax.broadcasted_iota(jnp.int32, (E, E), 0)
    ic = lax.broadcasted_iota(jnp.int32, (E, E), 1)
    u16 = (ir < ic).astype(jnp.float32)
    excl = lax.dot_general(nblk, u16, (((1,), (0,)), ((), ())),
                           preferred_element_type=jnp.float32)
    off = excl * B
    posf = jnp.sum(jnp.where(z, off + rank, 0.0), axis=1, keepdims=True)
    pos_ref[...] = posf.astype(jnp.int32)

    ones_col = jnp.ones((T, 1), jnp.float32)
    hist_c = lax.dot_general(zf32, ones_col, (((0,), (0,)), ((), ())),
                             preferred_element_type=jnp.float32)
    nblk_c = jnp.ceil(hist_c * (1.0 / B))
    l16 = (ir > ic).astype(jnp.float32)
    excl_c = lax.dot_general(l16, nblk_c, (((1,), (0,)), ((), ())),
                             preferred_element_type=jnp.float32)
    jb = lax.broadcasted_iota(jnp.int32, (E, NB), 1).astype(jnp.float32)
    ind = jnp.where((jb >= excl_c) & (jb < excl_c + nblk_c), 1.0, 0.0)
    e_vals = lax.broadcasted_iota(jnp.int32, (1, E), 1).astype(jnp.float32)
    be = lax.dot_general(e_vals, ind, (((1,), (0,)), ((), ())),
                         preferred_element_type=jnp.float32)
    be_ref[...] = be.astype(jnp.int32)


_route = pl.pallas_call(
    _route_body,
    out_shape=(
        jax.ShapeDtypeStruct((T, 1), jnp.int32),
        jax.ShapeDtypeStruct((1, NB), jnp.int32),
    ),
)


def _onehot(lanes, e):
    return 1 - jnp.minimum(jnp.abs(lanes - e), 1)


def _gather16(vec, idx):
    return lax.gather(
        vec, idx[:, None],
        lax.GatherDimensionNumbers(
            offset_dims=(), collapsed_slice_dims=(0,), start_index_map=(0,)),
        (1,), mode=lax.GatherScatterMode.PROMISE_IN_BOUNDS)


def _hist_chunk(seg_v, lo, lanes):
    h = jnp.zeros((16,), jnp.int32)
    for g in range(TPW // 16):
        sv = seg_v[pl.ds(lo + g * 16, 16)]
        for lane in range(16):
            h = h + _onehot(lanes, sv[lane])
    return h


@functools.cache
def _sc_kernels():
    mesh = plsc.VectorSubcoreMesh(core_axis_name="c", subcore_axis_name="s")

    @functools.partial(
        pl.kernel,
        out_type=(
            jax.ShapeDtypeStruct((TG, H), jnp.float32),
            jax.ShapeDtypeStruct((T,), jnp.int32),
            jax.ShapeDtypeStruct((NB,), jnp.int32),
        ),
        mesh=mesh,
        scratch_types=[
            pltpu.VMEM((2 * TPW,), jnp.int32),
            pltpu.VMEM((16,), jnp.int32),
            pltpu.VMEM((16,), jnp.int32),
            pltpu.VMEM((NW * 16,), jnp.int32),
            pltpu.VMEM((TPW,), jnp.int32),
            pltpu.VMEM((TPW, H), jnp.float32),
            pltpu.VMEM((NB,), jnp.int32),
            pltpu.VMEM_SHARED((NW * 16,), jnp.int32),
            pltpu.SemaphoreType.DMA,
            pltpu.SemaphoreType.DMA,
        ],
    )
    def sc_route_scatter(seg_hbm, x_hbm, xg_hbm, pos_hbm, be_hbm,
                         seg_v, ha_v, hb_v, hists_v, pos_v, rows_v, be_v,
                         board, sem, sem_x):
        cid = lax.axis_index("c")
        sid = lax.axis_index("s")
        wid = sid * NC + cid
        lanes = lax.broadcasted_iota(jnp.int32, (16,), 0)
        xcp = pltpu.async_copy(x_hbm.at[pl.ds(wid * TPW, TPW)], rows_v, sem_x)

        pltpu.sync_copy(seg_hbm.at[pl.ds(sid * 2 * TPW, 2 * TPW)], seg_v)
        ha_v[...] = _hist_chunk(seg_v, 0, lanes)
        hb_v[...] = _hist_chunk(seg_v, TPW, lanes)
        pltpu.sync_copy(ha_v, board.at[pl.ds((sid * 2) * 16, 16)])
        pltpu.sync_copy(hb_v, board.at[pl.ds((sid * 2 + 1) * 16, 16)])
        plsc.subcore_barrier()

        pltpu.sync_copy(board, hists_v)

        def acc(w, carry):
            tot, base = carry
            row = hists_v[pl.ds(w * 16, 16)]
            wf = jnp.full((16,), jnp.where(w < wid, 1, 0))
            return tot + row, base + row * wf

        total, tile_base = lax.fori_loop(
            0, NW, acc,
            (jnp.zeros((16,), jnp.int32), jnp.zeros((16,), jnp.int32)))
        nblk = (total + (B - 1)) >> LOG2B
        excl = jnp.zeros((16,), jnp.int32)
        run = jnp.int32(0)
        for e in range(E):
            excl = excl + _onehot(lanes, e) * run
            run = run + nblk[e]
        my_start = (excl << LOG2B) + tile_base

        cnt = jnp.zeros((16,), jnp.int32)
        for g in range(TPW // 16):
            sv = seg_v[pl.ds(cid * TPW + g * 16, 16)]
            gath = _gather16(my_start + cnt, sv)
            dup = jnp.zeros((16,), jnp.int32)
            for l2 in range(16):
                same = _onehot(sv, sv[l2])
                after = jnp.minimum(jnp.maximum(lanes - l2, 0), 1)
                dup = dup + same * after
                cnt = cnt + _onehot(lanes, sv[l2])
            pos_v[pl.ds(g * 16, 16)] = gath + dup
        pltpu.sync_copy(pos_v, pos_hbm.at[pl.ds(wid * TPW, TPW)])

        xcp.wait()
        pltpu.async_copy(rows_v, xg_hbm.at[pos_v], sem).wait()

        @pl.when(wid == 0)
        def _():
            for k in range(NB // 16):
                jv = lanes + k * 16
                bev = jnp.zeros((16,), jnp.int32)
                for e in range(E):
                    bev = bev + jnp.minimum(jnp.maximum(jv - excl[e] + 1, 0), 1)
                be_v[pl.ds(k * 16, 16)] = bev - 1
            pltpu.sync_copy(be_v, be_hbm)

    @functools.partial(
        pl.kernel,
        out_type=jax.ShapeDtypeStruct((T, H), jnp.float32),
        mesh=mesh,
        scratch_types=[
            pltpu.VMEM((TPW,), jnp.int32),
            pltpu.VMEM((TPW, H), jnp.float32),
            pltpu.SemaphoreType.DMA,
            pltpu.SemaphoreType.DMA,
        ],
    )
    def sc_gather(y_hbm, pos_hbm, out_hbm, pos_v, rows_v, sem, sem2):
        wid = lax.axis_index("s") * NC + lax.axis_index("c")
        base = wid * TPW
        hp = TPW // 2
        pltpu.sync_copy(pos_hbm.at[pl.ds(base, TPW)], pos_v)
        c1 = pltpu.async_copy(y_hbm.at[pos_v.at[pl.ds(0, hp)]],
                              rows_v.at[pl.ds(0, hp)], sem)
        c1.wait()
        c2 = pltpu.async_copy(y_hbm.at[pos_v.at[pl.ds(hp, hp)]],
                              rows_v.at[pl.ds(hp, hp)], sem)
        s1 = pltpu.async_copy(rows_v.at[pl.ds(0, hp)],
                              out_hbm.at[pl.ds(base, hp)], sem2)
        c2.wait()
        s1.wait()
        pltpu.sync_copy(rows_v.at[pl.ds(hp, hp)],
                        out_hbm.at[pl.ds(base + hp, hp)])

    return sc_route_scatter, sc_gather


def _one_mlp(x, w1, w2, b1, b2):
    xb = x.astype(jnp.bfloat16)
    hid = lax.dot_general(xb, w1.astype(jnp.bfloat16),
                          (((1,), (1,)), ((), ())),
                          preferred_element_type=jnp.float32)
    hid = jnp.maximum(hid + b1, 0.0).astype(jnp.bfloat16)
    y = lax.dot_general(hid, w2.astype(jnp.bfloat16),
                        (((1,), (1,)), ((), ())),
                        preferred_element_type=jnp.float32)
    return y + b2


PB = 4


def _mlp_body(be_ref, xg_ref, *refs):
    w_refs, y_ref = refs[:-1], refs[-1]
    for p in range(PB):
        w1_ref, w2_ref, b1_ref, b2_ref = w_refs[4 * p:4 * p + 4]
        y_ref[p * B:(p + 1) * B, :] = _one_mlp(
            xg_ref[p * B:(p + 1) * B, :], w1_ref[0], w2_ref[0],
            b1_ref[0], b2_ref[0])


def _w_specs(p):
    return [
        pl.BlockSpec((1, M, H), lambda j, be, p=p: (be[PB * j + p], 0, 0)),
        pl.BlockSpec((1, H, M), lambda j, be, p=p: (be[PB * j + p], 0, 0)),
        pl.BlockSpec((1, 1, M), lambda j, be, p=p: (be[PB * j + p], 0, 0)),
        pl.BlockSpec((1, 1, H), lambda j, be, p=p: (be[PB * j + p], 0, 0)),
    ]


_mlp = pl.pallas_call(
    _mlp_body,
    grid_spec=pltpu.PrefetchScalarGridSpec(
        num_scalar_prefetch=1,
        grid=(NB // PB,),
        in_specs=[pl.BlockSpec((PB * B, H), lambda j, be: (j, 0))]
        + [s for p in range(PB) for s in _w_specs(p)],
        out_specs=pl.BlockSpec((PB * B, H), lambda j, be: (j, 0)),
    ),
    out_shape=jax.ShapeDtypeStruct((TG, H), jnp.float32),
    compiler_params=pltpu.CompilerParams(
        dimension_semantics=("arbitrary",),
    ),
)


@jax.jit
def kernel(math_hidden_states, segment_ids, W1_matrices, W2_matrices, b1_bias, b2_bias):
    sc_route_scatter, sc_gather = _sc_kernels()
    xg, pos, be = sc_route_scatter(segment_ids, math_hidden_states)
    b1r = b1_bias.reshape(E, 1, M)
    b2r = b2_bias.reshape(E, 1, H)
    wargs = [a for _ in range(PB)
             for a in (W1_matrices, W2_matrices, b1r, b2r)]
    y = _mlp(be, xg, *wargs)
    return sc_gather(y, pos)


# speedup vs baseline: 1.0183x; 1.0183x over previous
import functools

import jax
import jax.numpy as jnp
from jax import lax
from jax.experimental import pallas as pl
from jax.experimental.pallas import tpu as pltpu
from jax.experimental.pallas import tpu_sc as plsc

T = 2048
H = 768
M = 128
E = 16

B = 128
LOG2B = B.bit_length() - 1
NB = T // B + E
TG = NB * B
NC = 2
NS = 16
NW = NC * NS
TPW = T // NW


def _onehot(lanes, e):
    return 1 - jnp.minimum(jnp.abs(lanes - e), 1)


def _gather16(vec, idx):
    return lax.gather(
        vec, idx[:, None],
        lax.GatherDimensionNumbers(
            offset_dims=(), collapsed_slice_dims=(0,), start_index_map=(0,)),
        (1,), mode=lax.GatherScatterMode.PROMISE_IN_BOUNDS)


def _hist_chunk(seg_v, lo, lanes):
    h = jnp.zeros((16,), jnp.int32)
    for g in range(TPW // 16):
        sv = seg_v[pl.ds(lo + g * 16, 16)]
        for lane in range(16):
            h = h + _onehot(lanes, sv[lane])
    return h


@functools.cache
def _sc_kernels():
    mesh = plsc.VectorSubcoreMesh(core_axis_name="c", subcore_axis_name="s")

    @functools.partial(
        pl.kernel,
        out_type=(
            jax.ShapeDtypeStruct((TG, H), jnp.float32),
            jax.ShapeDtypeStruct((T,), jnp.int32),
            jax.ShapeDtypeStruct((NB,), jnp.int32),
        ),
        mesh=mesh,
        scratch_types=[
            pltpu.VMEM((2 * TPW,), jnp.int32),
            pltpu.VMEM((16,), jnp.int32),
            pltpu.VMEM((16,), jnp.int32),
            pltpu.VMEM((NW * 16,), jnp.int32),
            pltpu.VMEM((TPW,), jnp.int32),
            pltpu.VMEM((TPW, H), jnp.float32),
            pltpu.VMEM((NB,), jnp.int32),
            pltpu.VMEM_SHARED((NW * 16,), jnp.int32),
            pltpu.SemaphoreType.DMA,
            pltpu.SemaphoreType.DMA,
        ],
    )
    def sc_route_scatter(seg_hbm, x_hbm, xg_hbm, pos_hbm, be_hbm,
                         seg_v, ha_v, hb_v, hists_v, pos_v, rows_v, be_v,
                         board, sem, sem_x):
        cid = lax.axis_index("c")
        sid = lax.axis_index("s")
        wid = sid * NC + cid
        lanes = lax.broadcasted_iota(jnp.int32, (16,), 0)
        xcp = pltpu.async_copy(x_hbm.at[pl.ds(wid * TPW, TPW)], rows_v, sem_x)

        pltpu.sync_copy(seg_hbm.at[pl.ds(sid * 2 * TPW, 2 * TPW)], seg_v)
        ha_v[...] = _hist_chunk(seg_v, 0, lanes)
        hb_v[...] = _hist_chunk(seg_v, TPW, lanes)
        pltpu.sync_copy(ha_v, board.at[pl.ds((sid * 2) * 16, 16)])
        pltpu.sync_copy(hb_v, board.at[pl.ds((sid * 2 + 1) * 16, 16)])
        plsc.subcore_barrier()

        pltpu.sync_copy(board, hists_v)

        def acc(w, carry):
            tot, base = carry
            row = hists_v[pl.ds(w * 16, 16)]
            wf = jnp.full((16,), jnp.where(w < wid, 1, 0))
            return tot + row, base + row * wf

        total, tile_base = lax.fori_loop(
            0, NW, acc,
            (jnp.zeros((16,), jnp.int32), jnp.zeros((16,), jnp.int32)))
        nblk = (total + (B - 1)) >> LOG2B
        excl = jnp.zeros((16,), jnp.int32)
        run = jnp.int32(0)
        for e in range(E):
            excl = excl + _onehot(lanes, e) * run
            run = run + nblk[e]
        my_start = (excl << LOG2B) + tile_base

        cnt = jnp.zeros((16,), jnp.int32)
        for g in range(TPW // 16):
            sv = seg_v[pl.ds(cid * TPW + g * 16, 16)]
            gath = _gather16(my_start + cnt, sv)
            dup = jnp.zeros((16,), jnp.int32)
            for l2 in range(16):
                same = _onehot(sv, sv[l2])
                after = jnp.minimum(jnp.maximum(lanes - l2, 0), 1)
                dup = dup + same * after
                cnt = cnt + _onehot(lanes, sv[l2])
            pos_v[pl.ds(g * 16, 16)] = gath + dup
        pltpu.sync_copy(pos_v, pos_hbm.at[pl.ds(wid * TPW, TPW)])

        xcp.wait()
        pltpu.async_copy(rows_v, xg_hbm.at[pos_v], sem).wait()

        @pl.when(wid == 0)
        def _():
            for k in range(NB // 16):
                jv = lanes + k * 16
                bev = jnp.zeros((16,), jnp.int32)
                for e in range(E):
                    bev = bev + jnp.minimum(jnp.maximum(jv - excl[e] + 1, 0), 1)
                be_v[pl.ds(k * 16, 16)] = bev - 1
            pltpu.sync_copy(be_v, be_hbm)

    @functools.partial(
        pl.kernel,
        out_type=jax.ShapeDtypeStruct((T, H), jnp.float32),
        mesh=mesh,
        scratch_types=[
            pltpu.VMEM((TPW,), jnp.int32),
            pltpu.VMEM((TPW, H), jnp.float32),
            pltpu.SemaphoreType.DMA,
            pltpu.SemaphoreType.DMA,
        ],
    )
    def sc_gather(y_hbm, pos_hbm, out_hbm, pos_v, rows_v, sem, sem2):
        del sem2
        wid = lax.axis_index("s") * NC + lax.axis_index("c")
        base = wid * TPW
        pltpu.sync_copy(pos_hbm.at[pl.ds(base, TPW)], pos_v)
        pltpu.async_copy(y_hbm.at[pos_v], rows_v, sem).wait()
        pltpu.sync_copy(rows_v, out_hbm.at[pl.ds(base, TPW)])

    return sc_route_scatter, sc_gather


def _one_mlp(x, w1, w2, b1, b2):
    xb = x.astype(jnp.bfloat16)
    hid = lax.dot_general(xb, w1.astype(jnp.bfloat16),
                          (((1,), (1,)), ((), ())),
                          preferred_element_type=jnp.float32)
    hid = jnp.maximum(hid + b1, 0.0).astype(jnp.bfloat16)
    y = lax.dot_general(hid, w2.astype(jnp.bfloat16),
                        (((1,), (1,)), ((), ())),
                        preferred_element_type=jnp.float32)
    return y + b2


PB = 4


def _mlp_body(be_ref, xg_ref, *refs):
    w_refs, y_ref = refs[:-1], refs[-1]
    for p in range(PB):
        w1_ref, w2_ref, b1_ref, b2_ref = w_refs[4 * p:4 * p + 4]
        y_ref[p * B:(p + 1) * B, :] = _one_mlp(
            xg_ref[p * B:(p + 1) * B, :], w1_ref[0], w2_ref[0],
            b1_ref[0], b2_ref[0])


def _w_specs(p):
    return [
        pl.BlockSpec((1, M, H), lambda j, be, p=p: (be[PB * j + p], 0, 0)),
        pl.BlockSpec((1, H, M), lambda j, be, p=p: (be[PB * j + p], 0, 0)),
        pl.BlockSpec((1, 1, M), lambda j, be, p=p: (be[PB * j + p], 0, 0)),
        pl.BlockSpec((1, 1, H), lambda j, be, p=p: (be[PB * j + p], 0, 0)),
    ]


_mlp = pl.pallas_call(
    _mlp_body,
    grid_spec=pltpu.PrefetchScalarGridSpec(
        num_scalar_prefetch=1,
        grid=(NB // PB,),
        in_specs=[pl.BlockSpec((PB * B, H), lambda j, be: (j, 0))]
        + [s for p in range(PB) for s in _w_specs(p)],
        out_specs=pl.BlockSpec((PB * B, H), lambda j, be: (j, 0)),
    ),
    out_shape=jax.ShapeDtypeStruct((TG, H), jnp.float32),
    compiler_params=pltpu.CompilerParams(
        dimension_semantics=("arbitrary",),
    ),
)


@jax.jit
def kernel(math_hidden_states, segment_ids, W1_matrices, W2_matrices, b1_bias, b2_bias):
    sc_route_scatter, sc_gather = _sc_kernels()
    xg, pos, be = sc_route_scatter(segment_ids, math_hidden_states)
    b1r = b1_bias.reshape(E, 1, M)
    b2r = b2_bias.reshape(E, 1, H)
    wargs = [a for _ in range(PB)
             for a in (W1_matrices, W2_matrices, b1r, b2r)]
    y = _mlp(be, xg, *wargs)
    return sc_gather(y, pos)
